# baseline (speedup 1.0000x reference)
import functools

import jax
import jax.numpy as jnp
from jax import lax
from jax.experimental import pallas as pl
from jax.experimental.pallas import tpu as pltpu
from jax.experimental.pallas import tpu_sc as plsc

D_MODEL = 1024
NUM_BANDS = 5
BATCH = 16384

NC, NS = 2, 16
NW = NC * NS
SPLIT = 4
PBATCH = BATCH // SPLIT
BPW = PBATCH // NW
BCH = 4
NGRP = BPW // (2 * BCH)
NBUF = 3
LANES = 16
NVEC = D_MODEL // LANES
G = 8

_LO = (0.5, 4.0, 8.0, 13.0, 30.0)
_HI = (4.0, 8.0, 13.0, 30.0, 100.0)


def _combine_body(emb_ref, wt_ref, b_ref, lo_ref, hi_ref, out_ref):
    w0 = wt_ref[0:1, :]
    w1 = wt_ref[1:2, :]
    out_ref[:, :] = (
        emb_ref[:, :] + lo_ref[:, :] * w0 + hi_ref[:, :] * w1
        + b_ref[:].reshape(1, D_MODEL)
    )


def _combine(emb_table, freq_wt, freq_b):
    lo = jnp.array(_LO, dtype=jnp.float32).reshape(NUM_BANDS, 1)
    hi = jnp.array(_HI, dtype=jnp.float32).reshape(NUM_BANDS, 1)
    return pl.pallas_call(
        _combine_body,
        out_shape=jax.ShapeDtypeStruct((NUM_BANDS, D_MODEL), jnp.float32),
    )(emb_table, freq_wt, freq_b, lo, hi)


_MESH = plsc.VectorSubcoreMesh(core_axis_name="c", subcore_axis_name="s")


def _copy_row(buf, tab_v, slot, bb, j, base):
    vs = [tab_v[pl.ds(base + u * LANES, LANES)] for u in range(G)]
    for k0 in range(G, NVEC + G, G):
        if k0 <= NVEC - G:
            nxt = [
                tab_v[pl.ds(base + (k0 + u) * LANES, LANES)]
                for u in range(G)
            ]
        else:
            nxt = None
        for u in range(G):
            buf[slot, bb, j, pl.ds((k0 - G + u) * LANES, LANES)] = vs[u]
        vs = nxt


@functools.partial(
    pl.kernel,
    out_type=jax.ShapeDtypeStruct((PBATCH, NUM_BANDS, D_MODEL), jnp.float32),
    mesh=_MESH,
    scratch_types=[
        pltpu.VMEM((NUM_BANDS * D_MODEL,), jnp.float32),
        pltpu.VMEM((BPW * NUM_BANDS,), jnp.int32),
        pltpu.VMEM((NBUF, BCH, NUM_BANDS, D_MODEL), jnp.float32),
        pltpu.SemaphoreType.DMA,
    ],
)
def _lookup(table_hbm, idx_hbm, out_hbm, tab_v, idx_v, buf, ssem):
    wid = lax.axis_index("s") * NC + lax.axis_index("c")
    bbase = wid * BPW
    GROWS = 2 * BCH * NUM_BANDS

    pltpu.sync_copy(table_hbm, tab_v)
    pltpu.sync_copy(
        idx_hbm.at[pl.ds(bbase * NUM_BANDS, BPW * NUM_BANDS)], idx_v
    )

    def wait_scatter():
        pltpu.make_async_copy(
            buf.at[0], out_hbm.at[pl.ds(0, BCH)], ssem
        ).wait()

    def group(g, _):
        o = g * GROWS
        iv0 = idx_v[pl.ds(o, LANES)]
        iv1 = idx_v[pl.ds(o + 16, LANES)]
        iv2 = idx_v[pl.ds(o + 24, LANES)]

        for h in range(2):
            t = 2 * g + h
            slot = lax.rem(t, NBUF)

            @pl.when(t >= NBUF)
            def _():
                wait_scatter()

            for r in range(BCH * NUM_BANDS):
                rr = h * BCH * NUM_BANDS + r
                if rr < 16:
                    i = iv0[rr]
                elif rr < 32:
                    i = iv1[rr - 16]
                else:
                    i = iv2[rr - 24]
                bb, j = divmod(r, NUM_BANDS)
                _copy_row(buf, tab_v, slot, bb, j, i * D_MODEL)

            pltpu.async_copy(
                buf.at[slot], out_hbm.at[pl.ds(bbase + t * BCH, BCH)], ssem
            )
        return 0

    lax.fori_loop(0, NGRP, group, 0)
    for _ in range(NBUF):
        wait_scatter()


def kernel(band_indices, emb_table, freq_w, freq_b):
    table = _combine(emb_table, freq_w.T, freq_b).reshape(NUM_BANDS * D_MODEL)
    idx = band_indices.reshape(SPLIT, PBATCH * NUM_BANDS)
    parts = [_lookup(table, idx[k]) for k in range(SPLIT)]
    return jnp.concatenate(parts, axis=0)


# speedup vs baseline: 1.9806x; 1.9806x over previous
import functools

import jax
import jax.numpy as jnp
from jax import lax
from jax.experimental import pallas as pl
from jax.experimental.pallas import tpu as pltpu
from jax.experimental.pallas import tpu_sc as plsc

D_MODEL = 1024
NUM_BANDS = 5
BATCH = 16384

NC, NS = 2, 16
NW = NC * NS
SC_BATCH = 8192
TC_BATCH = BATCH - SC_BATCH
BPW = SC_BATCH // NW
BCH = 8
NCHUNK = BPW // BCH
NBUF = 2
LANES = 16
NVEC = D_MODEL // LANES
ROWS = BCH * NUM_BANDS
CELEM = ROWS * D_MODEL
G = 8
BB = 128

_LO = (0.5, 4.0, 8.0, 13.0, 30.0)
_HI = (4.0, 8.0, 13.0, 30.0, 100.0)


def _combine_body(emb_ref, wt_ref, b_ref, lo_ref, hi_ref, out_ref):
    w0 = wt_ref[0:1, :]
    w1 = wt_ref[1:2, :]
    out_ref[:, :] = (
        emb_ref[:, :] + lo_ref[:, :] * w0 + hi_ref[:, :] * w1
        + b_ref[:].reshape(1, D_MODEL)
    )


def _combine(emb_table, freq_wt, freq_b):
    lo = jnp.array(_LO, dtype=jnp.float32).reshape(NUM_BANDS, 1)
    hi = jnp.array(_HI, dtype=jnp.float32).reshape(NUM_BANDS, 1)
    return pl.pallas_call(
        _combine_body,
        out_shape=jax.ShapeDtypeStruct((NUM_BANDS, D_MODEL), jnp.float32),
    )(emb_table, freq_wt, freq_b, lo, hi)


_MESH = plsc.VectorSubcoreMesh(core_axis_name="c", subcore_axis_name="s")


@functools.partial(
    pl.kernel,
    out_type=jax.ShapeDtypeStruct((SC_BATCH * NUM_BANDS * D_MODEL,), jnp.float32),
    mesh=_MESH,
    scratch_types=[
        pltpu.VMEM((NUM_BANDS * D_MODEL,), jnp.float32),
        pltpu.VMEM((BPW * NUM_BANDS,), jnp.int32),
        pltpu.VMEM((NBUF * CELEM,), jnp.float32),
        pltpu.SemaphoreType.DMA,
    ],
)
def _sc_lookup(table_hbm, idx_hbm, out_hbm, tab_v, idx_v, buf, ssem):
    wid = lax.axis_index("s") * NC + lax.axis_index("c")
    ebase = wid * BPW * NUM_BANDS * D_MODEL

    pltpu.sync_copy(table_hbm, tab_v)
    pltpu.sync_copy(
        idx_hbm.at[pl.ds(wid * BPW * NUM_BANDS, BPW * NUM_BANDS)], idx_v
    )

    def wait_scatter():
        pltpu.make_async_copy(
            buf.at[pl.ds(0, CELEM)], out_hbm.at[pl.ds(0, CELEM)], ssem
        ).wait()

    def chunk(t, _):
        @pl.when(t >= NBUF)
        def _():
            wait_scatter()

        soff = lax.rem(t, NBUF) * CELEM
        o = t * ROWS
        iv0 = idx_v[pl.ds(o, LANES)]
        iv1 = idx_v[pl.ds(o + 16, LANES)]
        iv2 = idx_v[pl.ds(o + 24, LANES)]

        for r in range(ROWS):
            if r < 16:
                i = iv0[r]
            elif r < 32:
                i = iv1[r - 16]
            else:
                i = iv2[r - 24]
            base = i * D_MODEL
            doff = soff + r * D_MODEL
            vs = [tab_v[pl.ds(base + u * LANES, LANES)] for u in range(G)]
            for k0 in range(G, NVEC + G, G):
                if k0 <= NVEC - G:
                    nxt = [
                        tab_v[pl.ds(base + (k0 + u) * LANES, LANES)]
                        for u in range(G)
                    ]
                else:
                    nxt = None
                for u in range(G):
                    buf[pl.ds(doff + (k0 - G + u) * LANES, LANES)] = vs[u]
                vs = nxt

        pltpu.async_copy(
            buf.at[pl.ds(soff, CELEM)],
            out_hbm.at[pl.ds(ebase + t * CELEM, CELEM)],
            ssem,
        )
        return 0

    lax.fori_loop(0, NCHUNK, chunk, 0)
    for _ in range(NBUF):
        wait_scatter()


def _select_body(idx_ref, tab_ref, out_ref):
    e = idx_ref[...]
    out = jnp.broadcast_to(
        tab_ref[0][None, None, :], (BB, NUM_BANDS, D_MODEL)
    )
    for i in range(1, NUM_BANDS):
        out = jnp.where(e == i, tab_ref[i][None, None, :], out)
    out_ref[...] = out


def _tc_select(table, idx_tc3):
    nsc = SC_BATCH // BB
    return pl.pallas_call(
        _select_body,
        grid=(TC_BATCH // BB,),
        in_specs=[
            pl.BlockSpec((BB, NUM_BANDS, 1), lambda b: (b, 0, 0)),
            pl.BlockSpec((NUM_BANDS, D_MODEL), lambda b: (0, 0)),
        ],
        out_specs=pl.BlockSpec(
            (BB, NUM_BANDS, D_MODEL), lambda b, nsc=nsc: (b + nsc, 0, 0)
        ),
        out_shape=jax.ShapeDtypeStruct((BATCH, NUM_BANDS, D_MODEL), jnp.float32),
    )(idx_tc3, table)


def _fill_body(flat_ref, full_ref, out_ref):
    del full_ref
    x = flat_ref[...]
    out_ref[...] = x.reshape(BB, NUM_BANDS, D_MODEL)


def _tc_fill(sc_flat, full):
    return pl.pallas_call(
        _fill_body,
        grid=(SC_BATCH // BB,),
        in_specs=[
            pl.BlockSpec((BB * NUM_BANDS * D_MODEL,), lambda b: (b,)),
            pl.BlockSpec(memory_space=pl.ANY),
        ],
        out_specs=pl.BlockSpec((BB, NUM_BANDS, D_MODEL), lambda b: (b, 0, 0)),
        out_shape=jax.ShapeDtypeStruct((BATCH, NUM_BANDS, D_MODEL), jnp.float32),
        input_output_aliases={1: 0},
    )(sc_flat, full)


def kernel(band_indices, emb_table, freq_w, freq_b):
    table = _combine(emb_table, freq_w.T, freq_b)
    tablef = table.reshape(NUM_BANDS * D_MODEL)
    idx_sc = band_indices[:SC_BATCH].reshape(SC_BATCH * NUM_BANDS)
    idx_tc3 = band_indices[SC_BATCH:].reshape(TC_BATCH, NUM_BANDS, 1)
    sc_flat = _sc_lookup(tablef, idx_sc)
    full = _tc_select(table, idx_tc3)
    return _tc_fill(sc_flat, full)
